# Initial kernel scaffold; baseline (speedup 1.0000x reference)
import jax
import jax.numpy as jnp
from jax.experimental import pallas as pl


def kernel(inputs, adj, Wg, bg, Wc, bc):
    raise NotImplementedError("write your pallas kernel here")


# speedup vs baseline: 2.8545x; 2.8545x over previous
import functools

import jax
import jax.numpy as jnp
from jax.experimental import pallas as pl


def _degree_body(adj_ij, adj_ji, dinv_ref, *, K, T, n):
    i = pl.program_id(0)
    j = pl.program_id(1)
    p = adj_ij[...]
    q = adj_ji[...]
    t = jnp.maximum(p, q.T)
    rows = jax.lax.broadcasted_iota(jnp.int32, (K, K), 0) + i * K
    cols = jax.lax.broadcasted_iota(jnp.int32, (K, K), 1) + j * K
    t = jnp.where((rows < n) & (cols < n), t, 0.0)
    rs = jnp.sum(t, axis=1, keepdims=True)

    @pl.when(j == 0)
    def _():
        dinv_ref[...] = rs

    @pl.when(j > 0)
    def _():
        dinv_ref[...] += rs

    @pl.when(j == T - 1)
    def _():
        d = dinv_ref[...]
        dinv_ref[...] = jnp.where(d > 0, 1.0 / jnp.sqrt(d), 0.0)


def _degree(adj, K, T, n):
    NP = K * T
    body = functools.partial(_degree_body, K=K, T=T, n=n)
    return pl.pallas_call(
        body,
        grid=(T, T),
        in_specs=[
            pl.BlockSpec((K, K), lambda i, j: (i, j)),
            pl.BlockSpec((K, K), lambda i, j: (j, i)),
        ],
        out_specs=pl.BlockSpec((K, 1), lambda i, j: (i, 0)),
        out_shape=jax.ShapeDtypeStruct((NP, 1), jnp.float32),
    )(adj, adj)


def _normalize_body(adj_ij, adj_ji, dinv_i_ref, dinv_j_ref, sn_ref, *, K, n):
    i = pl.program_id(0)
    j = pl.program_id(1)
    p = adj_ij[...]
    q = adj_ji[...]
    t = jnp.maximum(p, q.T)
    rows = jax.lax.broadcasted_iota(jnp.int32, (K, K), 0) + i * K
    cols = jax.lax.broadcasted_iota(jnp.int32, (K, K), 1) + j * K
    t = jnp.where((rows < n) & (cols < n), t, 0.0)
    s = -((t * dinv_i_ref[...]) * dinv_j_ref[...])
    sn_ref[...] = s.astype(jnp.bfloat16)


def _normalize(adj, dinv, dinv_row, K, T, n):
    NP = K * T
    body = functools.partial(_normalize_body, K=K, n=n)
    return pl.pallas_call(
        body,
        grid=(T, T),
        in_specs=[
            pl.BlockSpec((K, K), lambda i, j: (i, j)),
            pl.BlockSpec((K, K), lambda i, j: (j, i)),
            pl.BlockSpec((K, 1), lambda i, j: (i, 0)),
            pl.BlockSpec((1, K), lambda i, j: (0, j)),
        ],
        out_specs=pl.BlockSpec((K, K), lambda i, j: (i, j)),
        out_shape=jax.ShapeDtypeStruct((NP, NP), jnp.bfloat16),
    )(adj, adj, dinv, dinv_row)


def _matvec_body(a_ref, x_ref, out_ref):
    xb = x_ref[...].astype(jnp.bfloat16)
    out_ref[...] = jax.lax.dot_general(
        a_ref[...], xb, (((1,), (0,)), ((), ())),
        preferred_element_type=jnp.float32)


def _matvec(sn, x, K2):
    NP, B = x.shape
    T2 = NP // K2
    return pl.pallas_call(
        _matvec_body,
        grid=(T2,),
        in_specs=[
            pl.BlockSpec((K2, NP), lambda i: (i, 0)),
            pl.BlockSpec((NP, B), lambda i: (0, 0)),
        ],
        out_specs=pl.BlockSpec((K2, B), lambda i: (i, 0)),
        out_shape=jax.ShapeDtypeStruct((NP, B), jnp.float32),
    )(sn, x)


def _combine_body(x_ref, y1_ref, y2s_ref, co_ref, h_ref):
    def r(v):
        return v.astype(jnp.bfloat16).astype(jnp.float32)

    x = x_ref[...]
    y1 = y1_ref[...]
    y2 = 2.0 * y2s_ref[...] - x
    xb, y1b, y2b = r(x), r(y1), r(y2)
    co = r(co_ref[...])
    g = xb * co[0, 0] + y1b * co[0, 1] + y2b * co[0, 2] + co_ref[0, 3]
    c = xb * co[0, 4] + y1b * co[0, 5] + y2b * co[0, 6] + co_ref[0, 7]
    h_ref[...] = (1.0 - jax.nn.sigmoid(g)) * jnp.tanh(c)


def _combine(x, y1, y2s, co):
    NP, B = x.shape
    return pl.pallas_call(
        _combine_body,
        grid=(1,),
        in_specs=[
            pl.BlockSpec((NP, B), lambda i: (0, 0)),
            pl.BlockSpec((NP, B), lambda i: (0, 0)),
            pl.BlockSpec((NP, B), lambda i: (0, 0)),
            pl.BlockSpec((1, 8), lambda i: (0, 0)),
        ],
        out_specs=pl.BlockSpec((NP, B), lambda i: (0, 0)),
        out_shape=jax.ShapeDtypeStruct((NP, B), jnp.float32),
    )(x, y1, y2s, co)


def _run(inputs, adj, Wg, bg, Wc, bc, K, K2):
    n = adj.shape[0]
    T = -(-n // K)
    NP = K * T
    num_layers = Wg.shape[0]

    dinv = _degree(adj, K, T, n)
    sn = _normalize(adj, dinv, dinv.T, K, T, n)
    x = jnp.pad(inputs.T, ((0, NP - n), (0, 0)))
    hs = []
    for l in range(num_layers):
        y1 = _matvec(sn, x, K2)
        y2s = _matvec(sn, y1, K2)
        co = jnp.stack([
            Wg[l, 0, 1], Wg[l, 1, 1], Wg[l, 2, 1], bg[l, 1],
            Wc[l, 0, 0], Wc[l, 1, 0], Wc[l, 2, 0], bc[l, 0],
        ]).reshape(1, 8)
        x = _combine(x, y1, y2s, co)
        hs.append(x)

    out = x[:n, :].T
    states = jnp.stack([h[:n, :].T for h in hs], axis=0)
    return out, states


def kernel(inputs, adj, Wg, bg, Wc, bc):
    return _run(inputs, adj, Wg, bg, Wc, bc, K=1024, K2=512)
